# baseline (speedup 1.0000x reference)
import functools

import jax
import jax.numpy as jnp
from jax import lax
from jax.experimental import pallas as pl
from jax.experimental.pallas import tpu as pltpu
from jax.experimental.pallas import tpu_sc as plsc

N_NODES = 10000
N_EDGES = 320000
DIM = 128
HD = DIM // 2
NC, NS = 2, 16
NW = NC * NS
CH = 80
EPT = N_EDGES // NS
NCT = EPT // CH
RPT = N_NODES // NS
G16 = N_NODES // 16

_sc_mesh = plsc.VectorSubcoreMesh(
    core_axis_name="c", subcore_axis_name="s", num_cores=NC, num_subcores=NS)


def _deg_body(dst_hbm, ident_hbm, degp_hbm, dstv, ldeg, identv, csem, shdeg):
    cid = lax.axis_index("c")
    sid = lax.axis_index("s")
    wid = sid * NC + cid
    zero16 = jnp.zeros((16,), jnp.float32)
    ones16 = jnp.ones((16,), jnp.float32)

    def _zero(i, carry):
        ldeg[i] = zero16
        return carry

    lax.fori_loop(0, G16, _zero, 0)

    @pl.when(sid == 0)
    def _():
        pltpu.sync_copy(ldeg, shdeg)

    plsc.subcore_barrier()

    pltpu.sync_copy(dst_hbm.at[wid], dstv)
    pltpu.sync_copy(ident_hbm, identv)

    four16 = jnp.full((16,), 4, jnp.int32)
    fifteen16 = jnp.full((16,), 15, jnp.int32)

    def _acc(i, carry):
        idx = dstv[i]
        r = lax.shift_right_logical(idx, four16)
        c = lax.bitwise_and(idx, fifteen16)
        plsc.addupdate_scatter(ldeg, [r, c], ones16)
        return carry

    lax.fori_loop(0, (N_EDGES // NW) // 16, _acc, 0)

    for cc in range(5):
        pltpu.async_copy(ldeg.at[pl.ds(cc * 125, 125)],
                         shdeg.at[identv.at[cc]], csem, add=True)
    for cc in range(5):
        pltpu.make_async_copy(ldeg.at[pl.ds(cc * 125, 125)],
                              shdeg.at[identv.at[cc]], csem).wait()

    plsc.subcore_barrier()

    @pl.when(sid == 0)
    def _():
        pltpu.sync_copy(shdeg, degp_hbm.at[cid])


_deg_call = pl.kernel(
    _deg_body,
    out_type=jax.ShapeDtypeStruct((NC, G16, 16), jnp.float32),
    mesh=_sc_mesh,
    compiler_params=pltpu.CompilerParams(
        needs_layout_passes=False, use_tc_tiling_on_sc=False),
    scratch_types=[
        pltpu.VMEM((G16, 16), jnp.int32),
        pltpu.VMEM((G16, 16), jnp.float32),
        pltpu.VMEM((5, 125), jnp.int32),
        pltpu.SemaphoreType.DMA,
        pltpu.VMEM_SHARED((G16, 16), jnp.float32),
    ],
)

_CHUNKS = [(i * CH, CH) for i in range(7)] + [(7 * CH, RPT - 7 * CH)]


def _agg_body(hs2_hbm, src_hbm, dst_hbm, p_hbm,
              srcv, dstv, rows_a, rows_b,
              gsem_a, gsem_b, ssem_a, ssem_b, sh_table, shp):
    cid = lax.axis_index("c")
    sid = lax.axis_index("s")
    zero16 = jnp.zeros((16,), jnp.float32)
    base = sid * RPT

    def _zero(t, carry):
        rows_a[t // 4, pl.ds((t % 4) * 16, 16)] = zero16
        return carry

    lax.fori_loop(0, CH * 4, _zero, 0)
    for off, n in _CHUNKS:
        pltpu.async_copy(rows_a.at[pl.ds(0, n)],
                         shp.at[pl.ds(base + off, n)], ssem_a)
    for off, n in _CHUNKS:
        pltpu.make_async_copy(rows_a.at[pl.ds(0, n)],
                              shp.at[pl.ds(base + off, n)], ssem_a).wait()

    stg = (rows_a, rows_b)
    isem = (gsem_a, gsem_b)
    osem = (ssem_a, ssem_b)

    def _stage_in(c):
        off, n = _CHUNKS[c]
        return (hs2_hbm.at[cid, pl.ds(base + off, n)],
                stg[c & 1].at[pl.ds(0, n)], isem[c & 1])

    def _stage_out(c):
        off, n = _CHUNKS[c]
        return (stg[c & 1].at[pl.ds(0, n)],
                sh_table.at[pl.ds(base + off, n)], osem[c & 1])

    pltpu.async_copy(*_stage_in(0))
    for c in range(8):
        pltpu.make_async_copy(*_stage_in(c)).wait()
        if c < 7:
            if c >= 1:
                pltpu.make_async_copy(*_stage_out(c - 1)).wait()
            pltpu.async_copy(*_stage_in(c + 1))
        pltpu.async_copy(*_stage_out(c))
    pltpu.make_async_copy(*_stage_out(6)).wait()
    pltpu.make_async_copy(*_stage_out(7)).wait()

    plsc.subcore_barrier()

    pltpu.sync_copy(src_hbm.at[sid], srcv)
    pltpu.sync_copy(dst_hbm.at[sid], dstv)

    def _gather(j, buf, sem):
        return pltpu.async_copy(sh_table.at[srcv.at[j]], buf, sem)

    def _scatter(j, buf, sem):
        return pltpu.async_copy(buf, shp.at[dstv.at[j]], sem, add=True)

    def _wait_gather(j, buf, sem):
        pltpu.make_async_copy(sh_table.at[srcv.at[j]], buf, sem).wait()

    def _wait_scatter(j, buf, sem):
        pltpu.make_async_copy(buf, shp.at[dstv.at[j]], sem).wait()

    _gather(0, rows_a, gsem_a)

    def _step(t, carry):
        j0 = 2 * t
        j1 = j0 + 1
        _wait_gather(j0, rows_a, gsem_a)

        @pl.when(t > 0)
        def _():
            _wait_scatter(j1, rows_b, ssem_b)

        _gather(j1, rows_b, gsem_b)
        _scatter(j0, rows_a, ssem_a)
        _wait_gather(j1, rows_b, gsem_b)
        _wait_scatter(j0, rows_a, ssem_a)

        @pl.when(t < NCT // 2 - 1)
        def _():
            _gather(j0 + 2, rows_a, gsem_a)

        _scatter(j1, rows_b, ssem_b)
        return carry

    lax.fori_loop(0, NCT // 2, _step, 0)
    _wait_scatter(NCT - 1, rows_b, ssem_b)

    plsc.subcore_barrier()

    def _dump_in(c):
        off, n = _CHUNKS[c]
        return (shp.at[pl.ds(base + off, n)],
                stg[c & 1].at[pl.ds(0, n)], isem[c & 1])

    def _dump_out(c):
        off, n = _CHUNKS[c]
        return (stg[c & 1].at[pl.ds(0, n)],
                p_hbm.at[cid, pl.ds(base + off, n)], osem[c & 1])

    pltpu.async_copy(*_dump_in(0))
    for c in range(8):
        pltpu.make_async_copy(*_dump_in(c)).wait()
        if c < 7:
            if c >= 1:
                pltpu.make_async_copy(*_dump_out(c - 1)).wait()
            pltpu.async_copy(*_dump_in(c + 1))
        pltpu.async_copy(*_dump_out(c))
    pltpu.make_async_copy(*_dump_out(6)).wait()
    pltpu.make_async_copy(*_dump_out(7)).wait()


_agg_call = pl.kernel(
    _agg_body,
    out_type=jax.ShapeDtypeStruct((NC, N_NODES, HD), jnp.float32),
    mesh=_sc_mesh,
    compiler_params=pltpu.CompilerParams(use_tc_tiling_on_sc=False),
    scratch_types=[
        pltpu.VMEM((NCT, CH), jnp.int32),
        pltpu.VMEM((NCT, CH), jnp.int32),
        pltpu.VMEM((CH, HD), jnp.float32),
        pltpu.VMEM((CH, HD), jnp.float32),
        pltpu.SemaphoreType.DMA,
        pltpu.SemaphoreType.DMA,
        pltpu.SemaphoreType.DMA,
        pltpu.SemaphoreType.DMA,
        pltpu.VMEM_SHARED((N_NODES, HD), jnp.float32),
        pltpu.VMEM_SHARED((N_NODES, HD), jnp.float32),
    ],
)


def _lin_body(x_ref, wt_ref, degp_ref, hs2_ref):
    deg = degp_ref[0] + degp_ref[1] + 1.0
    dinv = lax.rsqrt(deg)
    h = jnp.dot(x_ref[...], wt_ref[...],
                preferred_element_type=jnp.float32) * dinv
    hs2_ref[0] = h[:, :HD]
    hs2_ref[1] = h[:, HD:]


def _fin_body(p_ref, hs_ref, degp_ref, b_ref, out_ref):
    dinv = lax.rsqrt(degp_ref[0] + degp_ref[1] + 1.0)
    left = dinv * (p_ref[0] + hs_ref[0]) + b_ref[:, :HD]
    right = dinv * (p_ref[1] + hs_ref[1]) + b_ref[:, HD:]
    out_ref[...] = jnp.concatenate([left, right], axis=1)


_ROWS_BLK = 1000
_GRID = N_NODES // _ROWS_BLK


@jax.jit
def kernel(x, edge_index, W, b):
    ei = edge_index.astype(jnp.int32)
    src = ei[0]
    dst = ei[1]
    dst_t = dst.reshape(NW, G16, 16)
    src_e = src.reshape(NS, NCT, CH)
    dst_e = dst.reshape(NS, NCT, CH)
    ident = jnp.arange(G16, dtype=jnp.int32).reshape(5, 125)

    degp = _deg_call(dst_t, ident)
    degp3 = degp.reshape(NC, N_NODES, 1)

    hs2 = pl.pallas_call(
        _lin_body,
        grid=(_GRID,),
        in_specs=[
            pl.BlockSpec((_ROWS_BLK, DIM), lambda i: (i, 0)),
            pl.BlockSpec((DIM, DIM), lambda i: (0, 0)),
            pl.BlockSpec((NC, _ROWS_BLK, 1), lambda i: (0, i, 0)),
        ],
        out_specs=pl.BlockSpec((NC, _ROWS_BLK, HD), lambda i: (0, i, 0)),
        out_shape=jax.ShapeDtypeStruct((NC, N_NODES, HD), jnp.float32),
    )(x, W.T, degp3)

    p2 = _agg_call(hs2, src_e, dst_e)

    out = pl.pallas_call(
        _fin_body,
        grid=(_GRID,),
        in_specs=[
            pl.BlockSpec((NC, _ROWS_BLK, HD), lambda i: (0, i, 0)),
            pl.BlockSpec((NC, _ROWS_BLK, HD), lambda i: (0, i, 0)),
            pl.BlockSpec((NC, _ROWS_BLK, 1), lambda i: (0, i, 0)),
            pl.BlockSpec((1, DIM), lambda i: (0, 0)),
        ],
        out_specs=pl.BlockSpec((_ROWS_BLK, DIM), lambda i: (i, 0)),
        out_shape=jax.ShapeDtypeStruct((N_NODES, DIM), jnp.float32),
    )(p2, hs2, degp3, b.reshape(1, DIM))
    return out


# speedup vs baseline: 1.3353x; 1.3353x over previous
import functools

import jax
import jax.numpy as jnp
from jax import lax
from jax.experimental import pallas as pl
from jax.experimental.pallas import tpu as pltpu
from jax.experimental.pallas import tpu_sc as plsc

N_NODES = 10000
N_EDGES = 320000
DIM = 128
NC, NS = 2, 16
NW = NC * NS
EPW = N_EDGES // NW
CH = 40
NCH = EPW // CH
RPT = N_NODES // NS
G16 = N_NODES // 16

_sc_mesh = plsc.VectorSubcoreMesh(
    core_axis_name="c", subcore_axis_name="s", num_cores=NC, num_subcores=NS)


def _deg_body(dst_hbm, ident_hbm, degp_hbm, dstv, ldeg, identv, csem, shdeg):
    cid = lax.axis_index("c")
    sid = lax.axis_index("s")
    wid = sid * NC + cid
    zero16 = jnp.zeros((16,), jnp.float32)
    ones16 = jnp.ones((16,), jnp.float32)

    def _zero(i, carry):
        ldeg[i] = zero16
        return carry

    lax.fori_loop(0, G16, _zero, 0)

    @pl.when(sid == 0)
    def _():
        pltpu.sync_copy(ldeg, shdeg)

    plsc.subcore_barrier()

    pltpu.sync_copy(dst_hbm.at[wid], dstv)
    pltpu.sync_copy(ident_hbm, identv)

    four16 = jnp.full((16,), 4, jnp.int32)
    fifteen16 = jnp.full((16,), 15, jnp.int32)

    def _acc(i, carry):
        idx = dstv[i]
        r = lax.shift_right_logical(idx, four16)
        c = lax.bitwise_and(idx, fifteen16)
        plsc.addupdate_scatter(ldeg, [r, c], ones16)
        return carry

    lax.fori_loop(0, EPW // 16, _acc, 0)

    for cc in range(5):
        pltpu.async_copy(ldeg.at[pl.ds(cc * 125, 125)],
                         shdeg.at[identv.at[cc]], csem, add=True)
    for cc in range(5):
        pltpu.make_async_copy(ldeg.at[pl.ds(cc * 125, 125)],
                              shdeg.at[identv.at[cc]], csem).wait()

    plsc.subcore_barrier()

    @pl.when(sid == 0)
    def _():
        pltpu.sync_copy(shdeg, degp_hbm.at[cid])


_deg_call = pl.kernel(
    _deg_body,
    out_type=jax.ShapeDtypeStruct((NC, G16, 16), jnp.float32),
    mesh=_sc_mesh,
    compiler_params=pltpu.CompilerParams(
        needs_layout_passes=False, use_tc_tiling_on_sc=False),
    scratch_types=[
        pltpu.VMEM((G16, 16), jnp.int32),
        pltpu.VMEM((G16, 16), jnp.float32),
        pltpu.VMEM((5, 125), jnp.int32),
        pltpu.SemaphoreType.DMA,
        pltpu.VMEM_SHARED((G16, 16), jnp.float32),
    ],
)


def _agg_body(hs_hbm, src_hbm, dst_hbm, p_hbm,
              srcv, dstv, rows_a, rows_b, rows_c, rows_d, rows_e,
              gsem_a, gsem_b, gsem_c, gsem_d, gsem_e,
              ssem_a, ssem_b, ssem_c, ssem_d, ssem_e, shp):
    cid = lax.axis_index("c")
    sid = lax.axis_index("s")
    wid = sid * NC + cid
    zero16 = jnp.zeros((16,), jnp.float32)

    def _zero(t, carry):
        rows_a[t // 8, pl.ds((t % 8) * 16, 16)] = zero16
        return carry

    lax.fori_loop(0, CH * 8, _zero, 0)
    for c5 in range(15):
        pltpu.async_copy(rows_a, shp.at[pl.ds(sid * RPT + c5 * CH, CH)],
                         ssem_a)
    pltpu.async_copy(rows_a.at[pl.ds(0, 25)],
                     shp.at[pl.ds(sid * RPT + 600, 25)], ssem_a)
    for c5 in range(15):
        pltpu.make_async_copy(rows_a,
                              shp.at[pl.ds(sid * RPT + c5 * CH, CH)],
                              ssem_a).wait()
    pltpu.make_async_copy(rows_a.at[pl.ds(0, 25)],
                          shp.at[pl.ds(sid * RPT + 600, 25)], ssem_a).wait()
    plsc.subcore_barrier()

    pltpu.sync_copy(src_hbm.at[wid], srcv)
    pltpu.sync_copy(dst_hbm.at[wid], dstv)

    def _gather(j, buf, sem):
        return pltpu.async_copy(hs_hbm.at[srcv.at[j]], buf, sem)

    def _scatter(j, buf, sem):
        return pltpu.async_copy(buf, shp.at[dstv.at[j]], sem, add=True)

    def _wait_gather(j, buf, sem):
        pltpu.make_async_copy(hs_hbm.at[srcv.at[j]], buf, sem).wait()

    def _wait_scatter(j, buf, sem):
        pltpu.make_async_copy(buf, shp.at[dstv.at[j]], sem).wait()

    bufs = (rows_a, rows_b, rows_c, rows_d, rows_e)
    gsems = (gsem_a, gsem_b, gsem_c, gsem_d, gsem_e)
    ssems = (ssem_a, ssem_b, ssem_c, ssem_d, ssem_e)
    for k in range(3):
        _gather(k, bufs[k], gsems[k])

    def _step(t, carry):
        j0 = 5 * t
        for k in range(5):
            j = j0 + k
            kk = (k + 3) % 5
            _wait_gather(j, bufs[k], gsems[k])

            @pl.when(j >= 2)
            def _():
                _wait_scatter(j - 2, bufs[kk], ssems[kk])

            @pl.when(j + 3 < NCH)
            def _():
                _gather(j + 3, bufs[kk], gsems[kk])

            _scatter(j, bufs[k], ssems[k])
        return carry

    lax.fori_loop(0, NCH // 5, _step, 0)
    _wait_scatter(NCH - 2, bufs[(NCH - 2) % 5], ssems[(NCH - 2) % 5])
    _wait_scatter(NCH - 1, bufs[(NCH - 1) % 5], ssems[(NCH - 1) % 5])

    plsc.subcore_barrier()
    stg = (rows_a.at[pl.ds(0, 25)], rows_b.at[pl.ds(0, 25)])
    isem = (gsem_a, gsem_b)
    osem = (ssem_a, ssem_b)

    def _off(c):
        return sid * RPT + c * 25

    pltpu.async_copy(shp.at[pl.ds(_off(0), 25)], stg[0], isem[0])
    for c in range(25):
        par = c & 1
        alt = par ^ 1
        pltpu.make_async_copy(
            shp.at[pl.ds(_off(c), 25)], stg[par], isem[par]).wait()
        if c < 24:
            if c >= 1:
                pltpu.make_async_copy(
                    stg[alt], p_hbm.at[cid, pl.ds(_off(c - 1), 25)],
                    osem[alt]).wait()
            pltpu.async_copy(shp.at[pl.ds(_off(c + 1), 25)], stg[alt],
                             isem[alt])
        pltpu.async_copy(stg[par], p_hbm.at[cid, pl.ds(_off(c), 25)],
                         osem[par])
    pltpu.make_async_copy(stg[1], p_hbm.at[cid, pl.ds(_off(23), 25)],
                          osem[1]).wait()
    pltpu.make_async_copy(stg[0], p_hbm.at[cid, pl.ds(_off(24), 25)],
                          osem[0]).wait()


_agg_call = pl.kernel(
    _agg_body,
    out_type=jax.ShapeDtypeStruct((NC, N_NODES, DIM), jnp.float32),
    mesh=_sc_mesh,
    compiler_params=pltpu.CompilerParams(use_tc_tiling_on_sc=False),
    scratch_types=[
        pltpu.VMEM((NCH, CH), jnp.int32),
        pltpu.VMEM((NCH, CH), jnp.int32),
        pltpu.VMEM((CH, DIM), jnp.float32),
        pltpu.VMEM((CH, DIM), jnp.float32),
        pltpu.VMEM((CH, DIM), jnp.float32),
        pltpu.VMEM((CH, DIM), jnp.float32),
        pltpu.VMEM((CH, DIM), jnp.float32),
        pltpu.SemaphoreType.DMA,
        pltpu.SemaphoreType.DMA,
        pltpu.SemaphoreType.DMA,
        pltpu.SemaphoreType.DMA,
        pltpu.SemaphoreType.DMA,
        pltpu.SemaphoreType.DMA,
        pltpu.SemaphoreType.DMA,
        pltpu.SemaphoreType.DMA,
        pltpu.SemaphoreType.DMA,
        pltpu.SemaphoreType.DMA,
        pltpu.VMEM_SHARED((N_NODES, DIM), jnp.float32),
    ],
)


def _lin_body(x_ref, wt_ref, degp_ref, hs_ref):
    deg = degp_ref[0] + degp_ref[1] + 1.0
    dinv = lax.rsqrt(deg)
    hs_ref[...] = jnp.dot(x_ref[...], wt_ref[...],
                          preferred_element_type=jnp.float32) * dinv


def _fin_body(p_ref, hs_ref, degp_ref, b_ref, out_ref):
    dinv = lax.rsqrt(degp_ref[0] + degp_ref[1] + 1.0)
    out_ref[...] = dinv * (p_ref[0] + p_ref[1] + hs_ref[...]) + b_ref[...]


_ROWS_BLK = 1000
_GRID = N_NODES // _ROWS_BLK


@jax.jit
def kernel(x, edge_index, W, b):
    ei = edge_index.astype(jnp.int32)
    src = ei[0]
    dst = ei[1]
    dst_t = dst.reshape(NW, G16, 16)
    src3 = src.reshape(NW, NCH, CH)
    dst3 = dst.reshape(NW, NCH, CH)
    ident = jnp.arange(G16, dtype=jnp.int32).reshape(5, 125)

    degp = _deg_call(dst_t, ident)
    degp3 = degp.reshape(NC, N_NODES, 1)

    hs = pl.pallas_call(
        _lin_body,
        grid=(_GRID,),
        in_specs=[
            pl.BlockSpec((_ROWS_BLK, DIM), lambda i: (i, 0)),
            pl.BlockSpec((DIM, DIM), lambda i: (0, 0)),
            pl.BlockSpec((NC, _ROWS_BLK, 1), lambda i: (0, i, 0)),
        ],
        out_specs=pl.BlockSpec((_ROWS_BLK, DIM), lambda i: (i, 0)),
        out_shape=jax.ShapeDtypeStruct((N_NODES, DIM), jnp.float32),
    )(x, W.T, degp3)

    p = _agg_call(hs, src3, dst3)

    out = pl.pallas_call(
        _fin_body,
        grid=(_GRID,),
        in_specs=[
            pl.BlockSpec((NC, _ROWS_BLK, DIM), lambda i: (0, i, 0)),
            pl.BlockSpec((_ROWS_BLK, DIM), lambda i: (i, 0)),
            pl.BlockSpec((NC, _ROWS_BLK, 1), lambda i: (0, i, 0)),
            pl.BlockSpec((1, DIM), lambda i: (0, 0)),
        ],
        out_specs=pl.BlockSpec((_ROWS_BLK, DIM), lambda i: (i, 0)),
        out_shape=jax.ShapeDtypeStruct((N_NODES, DIM), jnp.float32),
    )(p, hs, degp3, b.reshape(1, DIM))
    return out


# speedup vs baseline: 1.3463x; 1.0082x over previous
import functools

import jax
import jax.numpy as jnp
from jax import lax
from jax.experimental import pallas as pl
from jax.experimental.pallas import tpu as pltpu
from jax.experimental.pallas import tpu_sc as plsc

N_NODES = 10000
N_EDGES = 320000
DIM = 128
NC, NS = 2, 16
NW = NC * NS
EPW = N_EDGES // NW
CH = 40
NCH = EPW // CH
RPT = N_NODES // NS
G16 = N_NODES // 16

_sc_mesh = plsc.VectorSubcoreMesh(
    core_axis_name="c", subcore_axis_name="s", num_cores=NC, num_subcores=NS)


def _deg_body(dst_hbm, ident_hbm, degp_hbm, dstv, ldeg, identv, csem, shdeg):
    cid = lax.axis_index("c")
    sid = lax.axis_index("s")
    wid = sid * NC + cid
    zero16 = jnp.zeros((16,), jnp.float32)
    ones16 = jnp.ones((16,), jnp.float32)

    def _zero(i, carry):
        ldeg[i] = zero16
        return carry

    lax.fori_loop(0, G16, _zero, 0, unroll=4)

    @pl.when(sid == 0)
    def _():
        pltpu.sync_copy(ldeg, shdeg)

    plsc.subcore_barrier()

    pltpu.sync_copy(dst_hbm.at[wid], dstv)
    pltpu.sync_copy(ident_hbm, identv)

    four16 = jnp.full((16,), 4, jnp.int32)
    fifteen16 = jnp.full((16,), 15, jnp.int32)

    def _acc(i, carry):
        idx = dstv[i]
        r = lax.shift_right_logical(idx, four16)
        c = lax.bitwise_and(idx, fifteen16)
        plsc.addupdate_scatter(ldeg, [r, c], ones16)
        return carry

    lax.fori_loop(0, EPW // 16, _acc, 0, unroll=4)

    for cc in range(5):
        pltpu.async_copy(ldeg.at[pl.ds(cc * 125, 125)],
                         shdeg.at[identv.at[cc]], csem, add=True)
    for cc in range(5):
        pltpu.make_async_copy(ldeg.at[pl.ds(cc * 125, 125)],
                              shdeg.at[identv.at[cc]], csem).wait()

    plsc.subcore_barrier()

    @pl.when(sid == 0)
    def _():
        pltpu.sync_copy(shdeg, degp_hbm.at[cid])


_deg_call = pl.kernel(
    _deg_body,
    out_type=jax.ShapeDtypeStruct((NC, G16, 16), jnp.float32),
    mesh=_sc_mesh,
    compiler_params=pltpu.CompilerParams(
        needs_layout_passes=False, use_tc_tiling_on_sc=False),
    scratch_types=[
        pltpu.VMEM((G16, 16), jnp.int32),
        pltpu.VMEM((G16, 16), jnp.float32),
        pltpu.VMEM((5, 125), jnp.int32),
        pltpu.SemaphoreType.DMA,
        pltpu.VMEM_SHARED((G16, 16), jnp.float32),
    ],
)


def _agg_body(hs_hbm, src_hbm, dst_hbm, p_hbm,
              srcv, dstv, rows_a, rows_b, rows_c, rows_d, rows_e,
              gsem_a, gsem_b, gsem_c, gsem_d, gsem_e,
              ssem_a, ssem_b, ssem_c, ssem_d, ssem_e, shp):
    cid = lax.axis_index("c")
    sid = lax.axis_index("s")
    wid = sid * NC + cid
    zero16 = jnp.zeros((16,), jnp.float32)

    def _zero(t, carry):
        rows_a[t // 8, pl.ds((t % 8) * 16, 16)] = zero16
        return carry

    lax.fori_loop(0, CH * 8, _zero, 0)
    for c5 in range(15):
        pltpu.async_copy(rows_a, shp.at[pl.ds(sid * RPT + c5 * CH, CH)],
                         ssem_a)
    pltpu.async_copy(rows_a.at[pl.ds(0, 25)],
                     shp.at[pl.ds(sid * RPT + 600, 25)], ssem_a)
    for c5 in range(15):
        pltpu.make_async_copy(rows_a,
                              shp.at[pl.ds(sid * RPT + c5 * CH, CH)],
                              ssem_a).wait()
    pltpu.make_async_copy(rows_a.at[pl.ds(0, 25)],
                          shp.at[pl.ds(sid * RPT + 600, 25)], ssem_a).wait()
    plsc.subcore_barrier()

    pltpu.sync_copy(src_hbm.at[wid], srcv)
    pltpu.sync_copy(dst_hbm.at[wid], dstv)

    def _gather(j, buf, sem):
        return pltpu.async_copy(hs_hbm.at[srcv.at[j]], buf, sem)

    def _scatter(j, buf, sem):
        return pltpu.async_copy(buf, shp.at[dstv.at[j]], sem, add=True)

    def _wait_gather(j, buf, sem):
        pltpu.make_async_copy(hs_hbm.at[srcv.at[j]], buf, sem).wait()

    def _wait_scatter(j, buf, sem):
        pltpu.make_async_copy(buf, shp.at[dstv.at[j]], sem).wait()

    bufs = (rows_a, rows_b, rows_c, rows_d, rows_e)
    gsems = (gsem_a, gsem_b, gsem_c, gsem_d, gsem_e)
    ssems = (ssem_a, ssem_b, ssem_c, ssem_d, ssem_e)
    for k in range(3):
        _gather(k, bufs[k], gsems[k])

    def _step(t, carry):
        j0 = 5 * t
        for k in range(5):
            j = j0 + k
            kk = (k + 3) % 5
            _wait_gather(j, bufs[k], gsems[k])

            @pl.when(j >= 2)
            def _():
                _wait_scatter(j - 2, bufs[kk], ssems[kk])

            @pl.when(j + 3 < NCH)
            def _():
                _gather(j + 3, bufs[kk], gsems[kk])

            _scatter(j, bufs[k], ssems[k])
        return carry

    lax.fori_loop(0, NCH // 5, _step, 0)
    _wait_scatter(NCH - 2, bufs[(NCH - 2) % 5], ssems[(NCH - 2) % 5])
    _wait_scatter(NCH - 1, bufs[(NCH - 1) % 5], ssems[(NCH - 1) % 5])

    plsc.subcore_barrier()
    stg = (rows_a.at[pl.ds(0, 25)], rows_b.at[pl.ds(0, 25)])
    isem = (gsem_a, gsem_b)
    osem = (ssem_a, ssem_b)

    def _off(c):
        return sid * RPT + c * 25

    pltpu.async_copy(shp.at[pl.ds(_off(0), 25)], stg[0], isem[0])
    for c in range(25):
        par = c & 1
        alt = par ^ 1
        pltpu.make_async_copy(
            shp.at[pl.ds(_off(c), 25)], stg[par], isem[par]).wait()
        if c < 24:
            if c >= 1:
                pltpu.make_async_copy(
                    stg[alt], p_hbm.at[cid, pl.ds(_off(c - 1), 25)],
                    osem[alt]).wait()
            pltpu.async_copy(shp.at[pl.ds(_off(c + 1), 25)], stg[alt],
                             isem[alt])
        pltpu.async_copy(stg[par], p_hbm.at[cid, pl.ds(_off(c), 25)],
                         osem[par])
    pltpu.make_async_copy(stg[1], p_hbm.at[cid, pl.ds(_off(23), 25)],
                          osem[1]).wait()
    pltpu.make_async_copy(stg[0], p_hbm.at[cid, pl.ds(_off(24), 25)],
                          osem[0]).wait()


_agg_call = pl.kernel(
    _agg_body,
    out_type=jax.ShapeDtypeStruct((NC, N_NODES, DIM), jnp.float32),
    mesh=_sc_mesh,
    compiler_params=pltpu.CompilerParams(use_tc_tiling_on_sc=False),
    scratch_types=[
        pltpu.VMEM((NCH, CH), jnp.int32),
        pltpu.VMEM((NCH, CH), jnp.int32),
        pltpu.VMEM((CH, DIM), jnp.float32),
        pltpu.VMEM((CH, DIM), jnp.float32),
        pltpu.VMEM((CH, DIM), jnp.float32),
        pltpu.VMEM((CH, DIM), jnp.float32),
        pltpu.VMEM((CH, DIM), jnp.float32),
        pltpu.SemaphoreType.DMA,
        pltpu.SemaphoreType.DMA,
        pltpu.SemaphoreType.DMA,
        pltpu.SemaphoreType.DMA,
        pltpu.SemaphoreType.DMA,
        pltpu.SemaphoreType.DMA,
        pltpu.SemaphoreType.DMA,
        pltpu.SemaphoreType.DMA,
        pltpu.SemaphoreType.DMA,
        pltpu.SemaphoreType.DMA,
        pltpu.VMEM_SHARED((N_NODES, DIM), jnp.float32),
    ],
)


def _mm_body(x_ref, wt_ref, h_ref):
    h_ref[...] = jnp.dot(x_ref[...], wt_ref[...],
                         preferred_element_type=jnp.float32)


def _scale_body(h_ref, degp_ref, hs_ref):
    deg = degp_ref[0] + degp_ref[1] + 1.0
    dinv = lax.rsqrt(deg)
    hs_ref[...] = h_ref[...] * dinv


def _fin_body(p_ref, hs_ref, degp_ref, b_ref, out_ref):
    dinv = lax.rsqrt(degp_ref[0] + degp_ref[1] + 1.0)
    out_ref[...] = dinv * (p_ref[0] + p_ref[1] + hs_ref[...]) + b_ref[...]


_ROWS_BLK = 1000
_GRID = N_NODES // _ROWS_BLK


@jax.jit
def kernel(x, edge_index, W, b):
    ei = edge_index.astype(jnp.int32)
    src = ei[0]
    dst = ei[1]
    dst_t = dst.reshape(NW, G16, 16)
    src3 = src.reshape(NW, NCH, CH)
    dst3 = dst.reshape(NW, NCH, CH)
    ident = jnp.arange(G16, dtype=jnp.int32).reshape(5, 125)

    degp = _deg_call(dst_t, ident)
    degp3 = degp.reshape(NC, N_NODES, 1)

    h = pl.pallas_call(
        _mm_body,
        grid=(_GRID,),
        in_specs=[
            pl.BlockSpec((_ROWS_BLK, DIM), lambda i: (i, 0)),
            pl.BlockSpec((DIM, DIM), lambda i: (0, 0)),
        ],
        out_specs=pl.BlockSpec((_ROWS_BLK, DIM), lambda i: (i, 0)),
        out_shape=jax.ShapeDtypeStruct((N_NODES, DIM), jnp.float32),
    )(x, W.T)

    hs = pl.pallas_call(
        _scale_body,
        grid=(_GRID,),
        in_specs=[
            pl.BlockSpec((_ROWS_BLK, DIM), lambda i: (i, 0)),
            pl.BlockSpec((NC, _ROWS_BLK, 1), lambda i: (0, i, 0)),
        ],
        out_specs=pl.BlockSpec((_ROWS_BLK, DIM), lambda i: (i, 0)),
        out_shape=jax.ShapeDtypeStruct((N_NODES, DIM), jnp.float32),
    )(h, degp3)

    p = _agg_call(hs, src3, dst3)

    out = pl.pallas_call(
        _fin_body,
        grid=(_GRID,),
        in_specs=[
            pl.BlockSpec((NC, _ROWS_BLK, DIM), lambda i: (0, i, 0)),
            pl.BlockSpec((_ROWS_BLK, DIM), lambda i: (i, 0)),
            pl.BlockSpec((NC, _ROWS_BLK, 1), lambda i: (0, i, 0)),
            pl.BlockSpec((1, DIM), lambda i: (0, 0)),
        ],
        out_specs=pl.BlockSpec((_ROWS_BLK, DIM), lambda i: (i, 0)),
        out_shape=jax.ShapeDtypeStruct((N_NODES, DIM), jnp.float32),
    )(p, hs, degp3, b.reshape(1, DIM))
    return out


# speedup vs baseline: 1.4621x; 1.0860x over previous
import functools

import jax
import jax.numpy as jnp
from jax import lax
from jax.experimental import pallas as pl
from jax.experimental.pallas import tpu as pltpu
from jax.experimental.pallas import tpu_sc as plsc

N_NODES = 10000
N_EDGES = 320000
DIM = 128
NC, NS = 2, 16
NW = NC * NS
EPW = N_EDGES // NW
CH = 40
NCH = EPW // CH
RPT = N_NODES // NS
G16 = N_NODES // 16

_sc_mesh = plsc.VectorSubcoreMesh(
    core_axis_name="c", subcore_axis_name="s", num_cores=NC, num_subcores=NS)


def _deg_body(dst_hbm, ident_hbm, degp_hbm, dstv, ldeg, identv, csem, shdeg):
    cid = lax.axis_index("c")
    sid = lax.axis_index("s")
    wid = sid * NC + cid
    zero16 = jnp.zeros((16,), jnp.float32)
    ones16 = jnp.ones((16,), jnp.float32)

    def _zero(i, carry):
        ldeg[i] = zero16
        return carry

    lax.fori_loop(0, G16, _zero, 0, unroll=4)

    @pl.when(sid == 0)
    def _():
        pltpu.sync_copy(ldeg, shdeg)

    plsc.subcore_barrier()

    pltpu.sync_copy(dst_hbm.at[wid], dstv)
    pltpu.sync_copy(ident_hbm, identv)

    four16 = jnp.full((16,), 4, jnp.int32)
    fifteen16 = jnp.full((16,), 15, jnp.int32)

    def _acc(i, carry):
        idx = dstv[i]
        r = lax.shift_right_logical(idx, four16)
        c = lax.bitwise_and(idx, fifteen16)
        plsc.addupdate_scatter(ldeg, [r, c], ones16)
        return carry

    lax.fori_loop(0, EPW // 16, _acc, 0, unroll=4)

    for cc in range(5):
        pltpu.async_copy(ldeg.at[pl.ds(cc * 125, 125)],
                         shdeg.at[identv.at[cc]], csem, add=True)
    for cc in range(5):
        pltpu.make_async_copy(ldeg.at[pl.ds(cc * 125, 125)],
                              shdeg.at[identv.at[cc]], csem).wait()

    plsc.subcore_barrier()

    @pl.when(sid == 0)
    def _():
        pltpu.sync_copy(shdeg, degp_hbm.at[cid])


_deg_call = pl.kernel(
    _deg_body,
    out_type=jax.ShapeDtypeStruct((NC, G16, 16), jnp.float32),
    mesh=_sc_mesh,
    compiler_params=pltpu.CompilerParams(
        needs_layout_passes=False, use_tc_tiling_on_sc=False),
    scratch_types=[
        pltpu.VMEM((G16, 16), jnp.int32),
        pltpu.VMEM((G16, 16), jnp.float32),
        pltpu.VMEM((5, 125), jnp.int32),
        pltpu.SemaphoreType.DMA,
        pltpu.VMEM_SHARED((G16, 16), jnp.float32),
    ],
)


def _agg_body(hs_hbm, src_hbm, dst_hbm, p_hbm,
              srcv, dstv, rows_a, rows_b, rows_c, rows_d, rows_e,
              gsem_a, gsem_b, gsem_c, gsem_d, gsem_e,
              ssem_a, ssem_b, ssem_c, ssem_d, ssem_e, shp):
    cid = lax.axis_index("c")
    sid = lax.axis_index("s")
    wid = sid * NC + cid
    zero16 = jnp.zeros((16,), jnp.float32)

    def _zero(t, carry):
        rows_a[t // 8, pl.ds((t % 8) * 16, 16)] = zero16
        return carry

    lax.fori_loop(0, CH * 8, _zero, 0)
    for c5 in range(15):
        pltpu.async_copy(rows_a, shp.at[pl.ds(sid * RPT + c5 * CH, CH)],
                         ssem_a)
    pltpu.async_copy(rows_a.at[pl.ds(0, 25)],
                     shp.at[pl.ds(sid * RPT + 600, 25)], ssem_a)
    for c5 in range(15):
        pltpu.make_async_copy(rows_a,
                              shp.at[pl.ds(sid * RPT + c5 * CH, CH)],
                              ssem_a).wait()
    pltpu.make_async_copy(rows_a.at[pl.ds(0, 25)],
                          shp.at[pl.ds(sid * RPT + 600, 25)], ssem_a).wait()
    plsc.subcore_barrier()

    pltpu.sync_copy(src_hbm.at[wid], srcv)
    pltpu.sync_copy(dst_hbm.at[wid], dstv)

    def _gather(j, buf, sem):
        return pltpu.async_copy(hs_hbm.at[srcv.at[j]], buf, sem)

    def _scatter(j, buf, sem):
        return pltpu.async_copy(buf, shp.at[dstv.at[j]], sem, add=True)

    def _wait_gather(j, buf, sem):
        pltpu.make_async_copy(hs_hbm.at[srcv.at[j]], buf, sem).wait()

    def _wait_scatter(j, buf, sem):
        pltpu.make_async_copy(buf, shp.at[dstv.at[j]], sem).wait()

    bufs = (rows_a, rows_b, rows_c, rows_d, rows_e)
    gsems = (gsem_a, gsem_b, gsem_c, gsem_d, gsem_e)
    ssems = (ssem_a, ssem_b, ssem_c, ssem_d, ssem_e)
    for k in range(4):
        _gather(k, bufs[k], gsems[k])

    def _step(t, carry):
        j0 = 5 * t
        for k in range(5):
            j = j0 + k
            kk = (k + 4) % 5
            _wait_gather(j, bufs[k], gsems[k])

            @pl.when(j >= 1)
            def _():
                _wait_scatter(j - 1, bufs[kk], ssems[kk])

            @pl.when(j + 4 < NCH)
            def _():
                _gather(j + 4, bufs[kk], gsems[kk])

            _scatter(j, bufs[k], ssems[k])
        return carry

    lax.fori_loop(0, NCH // 5, _step, 0)
    _wait_scatter(NCH - 1, bufs[(NCH - 1) % 5], ssems[(NCH - 1) % 5])

    plsc.subcore_barrier()
    stg = (rows_a.at[pl.ds(0, 25)], rows_b.at[pl.ds(0, 25)])
    isem = (gsem_a, gsem_b)
    osem = (ssem_a, ssem_b)

    def _off(c):
        return sid * RPT + c * 25

    pltpu.async_copy(shp.at[pl.ds(_off(0), 25)], stg[0], isem[0])
    for c in range(25):
        par = c & 1
        alt = par ^ 1
        pltpu.make_async_copy(
            shp.at[pl.ds(_off(c), 25)], stg[par], isem[par]).wait()
        if c < 24:
            if c >= 1:
                pltpu.make_async_copy(
                    stg[alt], p_hbm.at[cid, pl.ds(_off(c - 1), 25)],
                    osem[alt]).wait()
            pltpu.async_copy(shp.at[pl.ds(_off(c + 1), 25)], stg[alt],
                             isem[alt])
        pltpu.async_copy(stg[par], p_hbm.at[cid, pl.ds(_off(c), 25)],
                         osem[par])
    pltpu.make_async_copy(stg[1], p_hbm.at[cid, pl.ds(_off(23), 25)],
                          osem[1]).wait()
    pltpu.make_async_copy(stg[0], p_hbm.at[cid, pl.ds(_off(24), 25)],
                          osem[0]).wait()


_agg_call = pl.kernel(
    _agg_body,
    out_type=jax.ShapeDtypeStruct((NC, N_NODES, DIM), jnp.float32),
    mesh=_sc_mesh,
    compiler_params=pltpu.CompilerParams(use_tc_tiling_on_sc=False),
    scratch_types=[
        pltpu.VMEM((NCH, CH), jnp.int32),
        pltpu.VMEM((NCH, CH), jnp.int32),
        pltpu.VMEM((CH, DIM), jnp.float32),
        pltpu.VMEM((CH, DIM), jnp.float32),
        pltpu.VMEM((CH, DIM), jnp.float32),
        pltpu.VMEM((CH, DIM), jnp.float32),
        pltpu.VMEM((CH, DIM), jnp.float32),
        pltpu.SemaphoreType.DMA,
        pltpu.SemaphoreType.DMA,
        pltpu.SemaphoreType.DMA,
        pltpu.SemaphoreType.DMA,
        pltpu.SemaphoreType.DMA,
        pltpu.SemaphoreType.DMA,
        pltpu.SemaphoreType.DMA,
        pltpu.SemaphoreType.DMA,
        pltpu.SemaphoreType.DMA,
        pltpu.SemaphoreType.DMA,
        pltpu.VMEM_SHARED((N_NODES, DIM), jnp.float32),
    ],
)


def _mm_body(x_ref, wt_ref, h_ref):
    h_ref[...] = jnp.dot(x_ref[...], wt_ref[...],
                         preferred_element_type=jnp.float32)


def _scale_body(h_ref, degp_ref, hs_ref):
    deg = degp_ref[0] + degp_ref[1] + 1.0
    dinv = lax.rsqrt(deg)
    hs_ref[...] = h_ref[...] * dinv


def _fin_body(p_ref, hs_ref, degp_ref, b_ref, out_ref):
    dinv = lax.rsqrt(degp_ref[0] + degp_ref[1] + 1.0)
    out_ref[...] = dinv * (p_ref[0] + p_ref[1] + hs_ref[...]) + b_ref[...]


_ROWS_BLK = 1000
_GRID = N_NODES // _ROWS_BLK


@jax.jit
def kernel(x, edge_index, W, b):
    ei = edge_index.astype(jnp.int32)
    src = ei[0]
    dst = ei[1]
    dst_t = dst.reshape(NW, G16, 16)
    src3 = src.reshape(NW, NCH, CH)
    dst3 = dst.reshape(NW, NCH, CH)
    ident = jnp.arange(G16, dtype=jnp.int32).reshape(5, 125)

    degp = _deg_call(dst_t, ident)
    degp3 = degp.reshape(NC, N_NODES, 1)

    h = pl.pallas_call(
        _mm_body,
        grid=(_GRID,),
        in_specs=[
            pl.BlockSpec((_ROWS_BLK, DIM), lambda i: (i, 0)),
            pl.BlockSpec((DIM, DIM), lambda i: (0, 0)),
        ],
        out_specs=pl.BlockSpec((_ROWS_BLK, DIM), lambda i: (i, 0)),
        out_shape=jax.ShapeDtypeStruct((N_NODES, DIM), jnp.float32),
    )(x, W.T)

    hs = pl.pallas_call(
        _scale_body,
        grid=(_GRID,),
        in_specs=[
            pl.BlockSpec((_ROWS_BLK, DIM), lambda i: (i, 0)),
            pl.BlockSpec((NC, _ROWS_BLK, 1), lambda i: (0, i, 0)),
        ],
        out_specs=pl.BlockSpec((_ROWS_BLK, DIM), lambda i: (i, 0)),
        out_shape=jax.ShapeDtypeStruct((N_NODES, DIM), jnp.float32),
    )(h, degp3)

    p = _agg_call(hs, src3, dst3)

    out = pl.pallas_call(
        _fin_body,
        grid=(_GRID,),
        in_specs=[
            pl.BlockSpec((NC, _ROWS_BLK, DIM), lambda i: (0, i, 0)),
            pl.BlockSpec((_ROWS_BLK, DIM), lambda i: (i, 0)),
            pl.BlockSpec((NC, _ROWS_BLK, 1), lambda i: (0, i, 0)),
            pl.BlockSpec((1, DIM), lambda i: (0, 0)),
        ],
        out_specs=pl.BlockSpec((_ROWS_BLK, DIM), lambda i: (i, 0)),
        out_shape=jax.ShapeDtypeStruct((N_NODES, DIM), jnp.float32),
    )(p, hs, degp3, b.reshape(1, DIM))
    return out


# speedup vs baseline: 1.4677x; 1.0038x over previous
import functools

import jax
import jax.numpy as jnp
from jax import lax
from jax.experimental import pallas as pl
from jax.experimental.pallas import tpu as pltpu
from jax.experimental.pallas import tpu_sc as plsc

N_NODES = 10000
N_EDGES = 320000
DIM = 128
NC, NS = 2, 16
NW = NC * NS
EPW = N_EDGES // NW
CH = 40
NCH = EPW // CH
RPT = N_NODES // NS
G16 = N_NODES // 16

_sc_mesh = plsc.VectorSubcoreMesh(
    core_axis_name="c", subcore_axis_name="s", num_cores=NC, num_subcores=NS)


def _deg_body(dst_hbm, ident_hbm, degp_hbm, dstv, ldeg, identv, csem, shdeg):
    cid = lax.axis_index("c")
    sid = lax.axis_index("s")
    wid = sid * NC + cid
    zero16 = jnp.zeros((16,), jnp.float32)
    ones16 = jnp.ones((16,), jnp.float32)

    def _zero(i, carry):
        ldeg[i] = zero16
        return carry

    lax.fori_loop(0, G16, _zero, 0, unroll=4)

    @pl.when(sid == 0)
    def _():
        pltpu.sync_copy(ldeg, shdeg)

    plsc.subcore_barrier()

    pltpu.sync_copy(dst_hbm.at[wid], dstv)
    pltpu.sync_copy(ident_hbm, identv)

    four16 = jnp.full((16,), 4, jnp.int32)
    fifteen16 = jnp.full((16,), 15, jnp.int32)

    def _acc(i, carry):
        idx = dstv[i]
        r = lax.shift_right_logical(idx, four16)
        c = lax.bitwise_and(idx, fifteen16)
        plsc.addupdate_scatter(ldeg, [r, c], ones16)
        return carry

    lax.fori_loop(0, EPW // 16, _acc, 0, unroll=4)

    for cc in range(5):
        pltpu.async_copy(ldeg.at[pl.ds(cc * 125, 125)],
                         shdeg.at[identv.at[cc]], csem, add=True)
    for cc in range(5):
        pltpu.make_async_copy(ldeg.at[pl.ds(cc * 125, 125)],
                              shdeg.at[identv.at[cc]], csem).wait()

    plsc.subcore_barrier()

    @pl.when(sid == 0)
    def _():
        pltpu.sync_copy(shdeg, degp_hbm.at[cid])


_deg_call = pl.kernel(
    _deg_body,
    out_type=jax.ShapeDtypeStruct((NC, G16, 16), jnp.float32),
    mesh=_sc_mesh,
    compiler_params=pltpu.CompilerParams(
        needs_layout_passes=False, use_tc_tiling_on_sc=False),
    scratch_types=[
        pltpu.VMEM((G16, 16), jnp.int32),
        pltpu.VMEM((G16, 16), jnp.float32),
        pltpu.VMEM((5, 125), jnp.int32),
        pltpu.SemaphoreType.DMA,
        pltpu.VMEM_SHARED((G16, 16), jnp.float32),
    ],
)


def _agg_body(hs_hbm, src_hbm, dst_hbm, p_hbm,
              srcv, dstv, rows_a, rows_b, rows_c, rows_d, rows_e, rows_f,
              gsem_a, gsem_b, gsem_c, gsem_d, gsem_e, gsem_f,
              ssem_a, ssem_b, ssem_c, ssem_d, ssem_e, ssem_f, shp):
    cid = lax.axis_index("c")
    sid = lax.axis_index("s")
    wid = sid * NC + cid
    zero16 = jnp.zeros((16,), jnp.float32)

    def _zero(t, carry):
        rows_a[t // 8, pl.ds((t % 8) * 16, 16)] = zero16
        return carry

    lax.fori_loop(0, CH * 8, _zero, 0)
    for c5 in range(15):
        pltpu.async_copy(rows_a, shp.at[pl.ds(sid * RPT + c5 * CH, CH)],
                         ssem_a)
    pltpu.async_copy(rows_a.at[pl.ds(0, 25)],
                     shp.at[pl.ds(sid * RPT + 600, 25)], ssem_a)
    for c5 in range(15):
        pltpu.make_async_copy(rows_a,
                              shp.at[pl.ds(sid * RPT + c5 * CH, CH)],
                              ssem_a).wait()
    pltpu.make_async_copy(rows_a.at[pl.ds(0, 25)],
                          shp.at[pl.ds(sid * RPT + 600, 25)], ssem_a).wait()
    plsc.subcore_barrier()

    pltpu.sync_copy(src_hbm.at[wid], srcv)
    pltpu.sync_copy(dst_hbm.at[wid], dstv)

    def _gather(j, buf, sem):
        return pltpu.async_copy(hs_hbm.at[srcv.at[j]], buf, sem)

    def _scatter(j, buf, sem):
        return pltpu.async_copy(buf, shp.at[dstv.at[j]], sem, add=True)

    def _wait_gather(j, buf, sem):
        pltpu.make_async_copy(hs_hbm.at[srcv.at[j]], buf, sem).wait()

    def _wait_scatter(j, buf, sem):
        pltpu.make_async_copy(buf, shp.at[dstv.at[j]], sem).wait()

    bufs = (rows_a, rows_b, rows_c, rows_d, rows_e, rows_f)
    gsems = (gsem_a, gsem_b, gsem_c, gsem_d, gsem_e, gsem_f)
    ssems = (ssem_a, ssem_b, ssem_c, ssem_d, ssem_e, ssem_f)
    for k in range(5):
        _gather(k, bufs[k], gsems[k])

    def _slot(j, k):
        kk = (k + 5) % 6
        _wait_gather(j, bufs[k], gsems[k])

        @pl.when(j >= 1)
        def _():
            _wait_scatter(j - 1, bufs[kk], ssems[kk])

        @pl.when(j + 5 < NCH)
        def _():
            _gather(j + 5, bufs[kk], gsems[kk])

        _scatter(j, bufs[k], ssems[k])

    def _step(t, carry):
        j0 = 6 * t
        for k in range(6):
            _slot(j0 + k, k)
        return carry

    _NF = NCH // 6 * 6
    lax.fori_loop(0, NCH // 6, _step, 0)
    for j in range(_NF, NCH):
        _slot(j, j % 6)
    _wait_scatter(NCH - 1, bufs[(NCH - 1) % 6], ssems[(NCH - 1) % 6])

    plsc.subcore_barrier()
    stg = (rows_a.at[pl.ds(0, 25)], rows_b.at[pl.ds(0, 25)])
    isem = (gsem_a, gsem_b)
    osem = (ssem_a, ssem_b)

    def _off(c):
        return sid * RPT + c * 25

    pltpu.async_copy(shp.at[pl.ds(_off(0), 25)], stg[0], isem[0])
    for c in range(25):
        par = c & 1
        alt = par ^ 1
        pltpu.make_async_copy(
            shp.at[pl.ds(_off(c), 25)], stg[par], isem[par]).wait()
        if c < 24:
            if c >= 1:
                pltpu.make_async_copy(
                    stg[alt], p_hbm.at[cid, pl.ds(_off(c - 1), 25)],
                    osem[alt]).wait()
            pltpu.async_copy(shp.at[pl.ds(_off(c + 1), 25)], stg[alt],
                             isem[alt])
        pltpu.async_copy(stg[par], p_hbm.at[cid, pl.ds(_off(c), 25)],
                         osem[par])
    pltpu.make_async_copy(stg[1], p_hbm.at[cid, pl.ds(_off(23), 25)],
                          osem[1]).wait()
    pltpu.make_async_copy(stg[0], p_hbm.at[cid, pl.ds(_off(24), 25)],
                          osem[0]).wait()


_agg_call = pl.kernel(
    _agg_body,
    out_type=jax.ShapeDtypeStruct((NC, N_NODES, DIM), jnp.float32),
    mesh=_sc_mesh,
    compiler_params=pltpu.CompilerParams(use_tc_tiling_on_sc=False),
    scratch_types=[
        pltpu.VMEM((NCH, CH), jnp.int32),
        pltpu.VMEM((NCH, CH), jnp.int32),
        pltpu.VMEM((CH, DIM), jnp.float32),
        pltpu.VMEM((CH, DIM), jnp.float32),
        pltpu.VMEM((CH, DIM), jnp.float32),
        pltpu.VMEM((CH, DIM), jnp.float32),
        pltpu.VMEM((CH, DIM), jnp.float32),
        pltpu.VMEM((CH, DIM), jnp.float32),
        pltpu.SemaphoreType.DMA,
        pltpu.SemaphoreType.DMA,
        pltpu.SemaphoreType.DMA,
        pltpu.SemaphoreType.DMA,
        pltpu.SemaphoreType.DMA,
        pltpu.SemaphoreType.DMA,
        pltpu.SemaphoreType.DMA,
        pltpu.SemaphoreType.DMA,
        pltpu.SemaphoreType.DMA,
        pltpu.SemaphoreType.DMA,
        pltpu.SemaphoreType.DMA,
        pltpu.SemaphoreType.DMA,
        pltpu.VMEM_SHARED((N_NODES, DIM), jnp.float32),
    ],
)


def _mm_body(x_ref, wt_ref, h_ref):
    h_ref[...] = jnp.dot(x_ref[...], wt_ref[...],
                         preferred_element_type=jnp.float32)


def _scale_body(h_ref, degp_ref, hs_ref):
    deg = degp_ref[0] + degp_ref[1] + 1.0
    dinv = lax.rsqrt(deg)
    hs_ref[...] = h_ref[...] * dinv


def _fin_body(p_ref, hs_ref, degp_ref, b_ref, out_ref):
    dinv = lax.rsqrt(degp_ref[0] + degp_ref[1] + 1.0)
    out_ref[...] = dinv * (p_ref[0] + p_ref[1] + hs_ref[...]) + b_ref[...]


_ROWS_BLK = 1000
_GRID = N_NODES // _ROWS_BLK


@jax.jit
def kernel(x, edge_index, W, b):
    ei = edge_index.astype(jnp.int32)
    src = ei[0]
    dst = ei[1]
    dst_t = dst.reshape(NW, G16, 16)
    src3 = src.reshape(NW, NCH, CH)
    dst3 = dst.reshape(NW, NCH, CH)
    ident = jnp.arange(G16, dtype=jnp.int32).reshape(5, 125)

    degp = _deg_call(dst_t, ident)
    degp3 = degp.reshape(NC, N_NODES, 1)

    h = pl.pallas_call(
        _mm_body,
        grid=(_GRID,),
        in_specs=[
            pl.BlockSpec((_ROWS_BLK, DIM), lambda i: (i, 0)),
            pl.BlockSpec((DIM, DIM), lambda i: (0, 0)),
        ],
        out_specs=pl.BlockSpec((_ROWS_BLK, DIM), lambda i: (i, 0)),
        out_shape=jax.ShapeDtypeStruct((N_NODES, DIM), jnp.float32),
    )(x, W.T)

    hs = pl.pallas_call(
        _scale_body,
        grid=(_GRID,),
        in_specs=[
            pl.BlockSpec((_ROWS_BLK, DIM), lambda i: (i, 0)),
            pl.BlockSpec((NC, _ROWS_BLK, 1), lambda i: (0, i, 0)),
        ],
        out_specs=pl.BlockSpec((_ROWS_BLK, DIM), lambda i: (i, 0)),
        out_shape=jax.ShapeDtypeStruct((N_NODES, DIM), jnp.float32),
    )(h, degp3)

    p = _agg_call(hs, src3, dst3)

    out = pl.pallas_call(
        _fin_body,
        grid=(_GRID,),
        in_specs=[
            pl.BlockSpec((NC, _ROWS_BLK, DIM), lambda i: (0, i, 0)),
            pl.BlockSpec((_ROWS_BLK, DIM), lambda i: (i, 0)),
            pl.BlockSpec((NC, _ROWS_BLK, 1), lambda i: (0, i, 0)),
            pl.BlockSpec((1, DIM), lambda i: (0, 0)),
        ],
        out_specs=pl.BlockSpec((_ROWS_BLK, DIM), lambda i: (i, 0)),
        out_shape=jax.ShapeDtypeStruct((N_NODES, DIM), jnp.float32),
    )(p, hs, degp3, b.reshape(1, DIM))
    return out
